# baseline (speedup 1.0000x reference)
import functools
import math

import jax
import jax.numpy as jnp
import numpy as np
from jax import lax
from jax.experimental import pallas as pl
from jax.experimental.pallas import tpu as pltpu
from jax.experimental.pallas import tpu_sc as plsc

VOCAB = 1000
D_MODEL = 128
MAX_LEN = 512
SEQ = 200
SCALE = float(math.sqrt(D_MODEL))

CHUNK = 128
NBUF = 4
REP = 32
DPAIR = D_MODEL // 2


def _pe_table() -> np.ndarray:
    pe = np.zeros((MAX_LEN, D_MODEL), dtype=np.float32)
    position = np.arange(0, MAX_LEN, dtype=np.float32)[:, None]
    div_term = np.exp(
        np.arange(0, D_MODEL, 2, dtype=np.float32) * (-math.log(10000.0) / D_MODEL)
    )
    pe[:, 0::2] = np.sin(position * div_term)
    pe[:, 1::2] = np.cos(position * div_term)
    return pe[:SEQ]


def _pack_halves(rows: jnp.ndarray) -> jnp.ndarray:
    bf = rows.astype(jnp.bfloat16)
    pairs = jnp.stack([bf[:, :DPAIR], bf[:, DPAIR:]], axis=-1)
    return jax.lax.bitcast_convert_type(pairs, jnp.int32)


def _make_sc_call(batch: int, seq: int):
    info = plsc.get_sparse_core_info()
    nc, ns = info.num_cores, info.num_subcores
    nw = nc * ns
    ntok = batch * seq
    assert ntok % (nw * CHUNK) == 0
    tok_per_w = ntok // nw
    nchunk = tok_per_w // CHUNK
    pe_rows = seq + CHUNK

    mesh = plsc.VectorSubcoreMesh(core_axis_name="c", subcore_axis_name="s")

    @functools.partial(
        pl.kernel,
        mesh=mesh,
        out_type=jax.ShapeDtypeStruct((ntok * D_MODEL,), jnp.float32),
        scratch_types=[
            pltpu.VMEM((tok_per_w,), jnp.int32),
            pltpu.VMEM((pe_rows * DPAIR,), jnp.int32),
        ]
        + [pltpu.VMEM((CHUNK, DPAIR), jnp.int32) for _ in range(NBUF)]
        + [pltpu.VMEM((CHUNK * D_MODEL,), jnp.float32) for _ in range(2)]
        + [pltpu.SemaphoreType.DMA] * (NBUF + 2),
        compiler_params=pltpu.CompilerParams(
            needs_layout_passes=False, use_tc_tiling_on_sc=False
        ),
    )
    def sc_embed(x_hbm, table_hbm, pe_hbm, out_hbm, idx_v, pe_v, *rest):
        rows = rest[:NBUF]
        obuf = rest[NBUF : NBUF + 2]
        gsem = rest[NBUF + 2 : 2 * NBUF + 2]
        osem = rest[2 * NBUF + 2 :]
        wid = lax.axis_index("s") * nc + lax.axis_index("c")
        tok0 = wid * tok_per_w
        pltpu.sync_copy(pe_hbm, pe_v)
        pltpu.sync_copy(x_hbm.at[pl.ds(tok0, tok_per_w)], idx_v)

        roff = (lax.iota(jnp.int32, 16) % REP) * VOCAB

        @plsc.parallel_loop(0, tok_per_w // 16, 1, unroll=8)
        def idx_adj(v):
            sl = pl.ds(v * 16, 16)
            idx_v[sl] = idx_v[sl] + roff

        def start_gather(c, b):
            pltpu.async_copy(
                table_hbm.at[idx_v.at[pl.ds(c * CHUNK, CHUNK)]], rows[b], gsem[b]
            )

        def wait_gather(c, b):
            pltpu.make_async_copy(
                table_hbm.at[idx_v.at[pl.ds(c * CHUNK, CHUNK)]], rows[b], gsem[b]
            ).wait()

        def start_scatter(c, b):
            pltpu.async_copy(
                obuf[b],
                out_hbm.at[pl.ds((tok0 + c * CHUNK) * D_MODEL, CHUNK * D_MODEL)],
                osem[b],
            )

        def wait_scatter(c, b):
            pltpu.make_async_copy(
                obuf[b],
                out_hbm.at[pl.ds((tok0 + c * CHUNK) * D_MODEL, CHUNK * D_MODEL)],
                osem[b],
            ).wait()

        def compute(c, br, bo):
            pe0 = lax.rem(c * CHUNK, seq)

            @plsc.parallel_loop(0, CHUNK, 1, unroll=2)
            def tok_body(t):
                for j in range(DPAIR // 16):
                    sl = pl.ds(j * 16, 16)
                    tv = rows[br][t, sl]
                    pv = pe_v[pl.ds((pe0 + t) * DPAIR + j * 16, 16)]
                    ta, tb = plsc.unpack(
                        plsc.bitcast(tv, jnp.bfloat16),
                        format=plsc.PackFormat.INTERLEAVED,
                    )
                    pa, pb = plsc.unpack(
                        plsc.bitcast(pv, jnp.bfloat16),
                        format=plsc.PackFormat.INTERLEAVED,
                    )
                    lo = t * D_MODEL + j * 16
                    obuf[bo][pl.ds(lo, 16)] = ta * SCALE + pa
                    obuf[bo][pl.ds(lo + DPAIR, 16)] = tb * SCALE + pb

        def stage(c, br, bo, owait):
            wait_gather(c, br)
            start_gather(c + 2, (br + 2) % NBUF)
            if owait:
                wait_scatter(c - 2, bo)
            compute(c, br, bo)
            start_scatter(c, bo)

        start_gather(0, 0)
        start_gather(1, 1)

        for c in range(NBUF):
            stage(c, c % NBUF, c % 2, owait=(c >= 2))

        def main_body(k, carry):
            c0 = k * NBUF
            for b in range(NBUF):
                stage(c0 + b, b, b % 2, owait=True)
            return carry

        lax.fori_loop(1, (nchunk - 2) // NBUF, main_body, 0)

        for c in range(nchunk - 2, nchunk):
            wait_gather(c, c % NBUF)
            wait_scatter(c - 2, c % 2)
            compute(c, c % NBUF, c % 2)
            start_scatter(c, c % 2)

        for c in range(nchunk - 2, nchunk):
            wait_scatter(c, c % 2)

    return sc_embed


def kernel(x, table):
    batch, seq = x.shape
    x_flat = x.reshape(-1).astype(jnp.int32)
    table_pack = _pack_halves(table)
    table_rep = jnp.tile(table_pack, (REP, 1))
    pe = _pe_table()
    pe_ext = jnp.asarray(np.concatenate([pe, pe[:CHUNK]], axis=0))
    pe_pack = _pack_halves(pe_ext).reshape(-1)
    sc_embed = _make_sc_call(batch, seq)
    out = sc_embed(x_flat, table_rep, pe_pack)
    return out.reshape(batch, seq, D_MODEL)


# speedup vs baseline: 1.0915x; 1.0915x over previous
import functools
import math

import jax
import jax.numpy as jnp
import numpy as np
from jax import lax
from jax.experimental import pallas as pl
from jax.experimental.pallas import tpu as pltpu
from jax.experimental.pallas import tpu_sc as plsc

VOCAB = 1000
D_MODEL = 128
MAX_LEN = 512
SEQ = 200
SCALE = float(math.sqrt(D_MODEL))

CHUNK = 128
NBUF = 4
REP = 16
DPAIR = D_MODEL // 2


def _pe_table() -> np.ndarray:
    pe = np.zeros((MAX_LEN, D_MODEL), dtype=np.float32)
    position = np.arange(0, MAX_LEN, dtype=np.float32)[:, None]
    div_term = np.exp(
        np.arange(0, D_MODEL, 2, dtype=np.float32) * (-math.log(10000.0) / D_MODEL)
    )
    pe[:, 0::2] = np.sin(position * div_term)
    pe[:, 1::2] = np.cos(position * div_term)
    return pe[:SEQ]


def _pack_halves(rows: jnp.ndarray) -> jnp.ndarray:
    bf = rows.astype(jnp.bfloat16)
    pairs = jnp.stack([bf[:, :DPAIR], bf[:, DPAIR:]], axis=-1)
    return jax.lax.bitcast_convert_type(pairs, jnp.int32)


def _make_sc_call(batch: int, seq: int):
    info = plsc.get_sparse_core_info()
    nc, ns = info.num_cores, info.num_subcores
    nw = nc * ns
    ntok = batch * seq
    assert ntok % (nw * CHUNK) == 0
    tok_per_w = ntok // nw
    nchunk = tok_per_w // CHUNK
    pe_rows = seq + CHUNK

    mesh = plsc.VectorSubcoreMesh(core_axis_name="c", subcore_axis_name="s")

    @functools.partial(
        pl.kernel,
        mesh=mesh,
        out_type=jax.ShapeDtypeStruct((ntok * D_MODEL,), jnp.float32),
        scratch_types=[
            pltpu.VMEM((tok_per_w,), jnp.int32),
            pltpu.VMEM((pe_rows * DPAIR,), jnp.int32),
        ]
        + [pltpu.VMEM((CHUNK, DPAIR), jnp.int32) for _ in range(NBUF)]
        + [pltpu.VMEM((CHUNK * D_MODEL,), jnp.float32) for _ in range(2)]
        + [pltpu.SemaphoreType.DMA] * (NBUF + 2),
        compiler_params=pltpu.CompilerParams(
            needs_layout_passes=False, use_tc_tiling_on_sc=False
        ),
    )
    def sc_embed(x_hbm, table_hbm, pe_hbm, out_hbm, idx_v, pe_v, *rest):
        rows = rest[:NBUF]
        obuf = rest[NBUF : NBUF + 2]
        gsem = rest[NBUF + 2 : 2 * NBUF + 2]
        osem = rest[2 * NBUF + 2 :]
        wid = lax.axis_index("s") * nc + lax.axis_index("c")
        tok0 = wid * tok_per_w
        pltpu.sync_copy(pe_hbm, pe_v)
        pltpu.sync_copy(x_hbm.at[pl.ds(tok0, tok_per_w)], idx_v)

        roff = (lax.iota(jnp.int32, 16) % REP) * VOCAB

        @plsc.parallel_loop(0, tok_per_w // 16, 1, unroll=8)
        def idx_adj(v):
            sl = pl.ds(v * 16, 16)
            idx_v[sl] = idx_v[sl] + roff

        def start_gather(c, b):
            pltpu.async_copy(
                table_hbm.at[idx_v.at[pl.ds(c * CHUNK, CHUNK)]], rows[b], gsem[b]
            )

        def wait_gather(c, b):
            pltpu.make_async_copy(
                table_hbm.at[idx_v.at[pl.ds(c * CHUNK, CHUNK)]], rows[b], gsem[b]
            ).wait()

        def start_scatter(c, b):
            pltpu.async_copy(
                obuf[b],
                out_hbm.at[pl.ds((tok0 + c * CHUNK) * D_MODEL, CHUNK * D_MODEL)],
                osem[b],
            )

        def wait_scatter(c, b):
            pltpu.make_async_copy(
                obuf[b],
                out_hbm.at[pl.ds((tok0 + c * CHUNK) * D_MODEL, CHUNK * D_MODEL)],
                osem[b],
            ).wait()

        def compute(c, br, bo):
            pe0 = lax.rem(c * CHUNK, seq)

            @plsc.parallel_loop(0, CHUNK, 1, unroll=2)
            def tok_body(t):
                for j in range(DPAIR // 16):
                    sl = pl.ds(j * 16, 16)
                    tv = rows[br][t, sl]
                    pv = pe_v[pl.ds((pe0 + t) * DPAIR + j * 16, 16)]
                    ta, tb = plsc.unpack(
                        plsc.bitcast(tv, jnp.bfloat16),
                        format=plsc.PackFormat.INTERLEAVED,
                    )
                    pa, pb = plsc.unpack(
                        plsc.bitcast(pv, jnp.bfloat16),
                        format=plsc.PackFormat.INTERLEAVED,
                    )
                    lo = t * D_MODEL + j * 16
                    obuf[bo][pl.ds(lo, 16)] = ta * SCALE + pa
                    obuf[bo][pl.ds(lo + DPAIR, 16)] = tb * SCALE + pb

        def stage(c, br, bo, owait):
            wait_gather(c, br)
            start_gather(c + 2, (br + 2) % NBUF)
            if owait:
                wait_scatter(c - 2, bo)
            compute(c, br, bo)
            start_scatter(c, bo)

        start_gather(0, 0)
        start_gather(1, 1)

        for c in range(NBUF):
            stage(c, c % NBUF, c % 2, owait=(c >= 2))

        def main_body(k, carry):
            c0 = k * NBUF
            for b in range(NBUF):
                stage(c0 + b, b, b % 2, owait=True)
            return carry

        lax.fori_loop(1, (nchunk - 2) // NBUF, main_body, 0)

        for c in range(nchunk - 2, nchunk):
            wait_gather(c, c % NBUF)
            wait_scatter(c - 2, c % 2)
            compute(c, c % NBUF, c % 2)
            start_scatter(c, c % 2)

        for c in range(nchunk - 2, nchunk):
            wait_scatter(c, c % 2)

    return sc_embed


def kernel(x, table):
    batch, seq = x.shape
    x_flat = x.reshape(-1).astype(jnp.int32)
    table_pack = _pack_halves(table)
    table_rep = jnp.tile(table_pack, (REP, 1))
    pe = _pe_table()
    pe_ext = jnp.asarray(np.concatenate([pe, pe[:CHUNK]], axis=0))
    pe_pack = _pack_halves(pe_ext).reshape(-1)
    sc_embed = _make_sc_call(batch, seq)
    out = sc_embed(x_flat, table_rep, pe_pack)
    return out.reshape(batch, seq, D_MODEL)
